# baseline (speedup 1.0000x reference)
import functools
import math

import jax
import jax.numpy as jnp
from jax import lax
from jax.experimental import pallas as pl
from jax.experimental.pallas import tpu as pltpu
from jax.experimental.pallas import tpu_sc as plsc

_NC = 2
_NS = 16
_NW = _NC * _NS

_LP = 16
_NEG = -1e30


def _gather_body(CW, nchunks, EW, hv, cid, did, gc, gd, idxc, idxd, rowsc, rowsd, semc, semd):
    wid = lax.axis_index("s") * _NC + lax.axis_index("c")
    base0 = wid * EW

    def fetch(i, b):
        base = base0 + i * CW
        pltpu.sync_copy(cid.at[pl.ds(base, CW)], idxc.at[b])
        pltpu.sync_copy(did.at[pl.ds(base, CW)], idxd.at[b])
        pltpu.async_copy(hv.at[idxc.at[b]], rowsc.at[b], semc.at[b])
        pltpu.async_copy(hv.at[idxd.at[b]], rowsd.at[b], semd.at[b])

    fetch(0, 0)

    def body(i, carry):
        b = lax.rem(i, 2)
        @pl.when(i + 1 < nchunks)
        def _():
            fetch(i + 1, 1 - b)

        pltpu.make_async_copy(hv.at[idxc.at[b]], rowsc.at[b], semc.at[b]).wait()
        pltpu.make_async_copy(hv.at[idxd.at[b]], rowsd.at[b], semd.at[b]).wait()
        base = base0 + i * CW
        pltpu.sync_copy(rowsc.at[b], gc.at[pl.ds(base, CW)])
        pltpu.sync_copy(rowsd.at[b], gd.at[pl.ds(base, CW)])
        return carry

    lax.fori_loop(0, nchunks, body, 0)


def _make_gather(E, N, H, dtype):
    CW = 80
    EW = E // _NW
    nchunks = EW // CW
    mesh = plsc.VectorSubcoreMesh(core_axis_name="c", subcore_axis_name="s")
    return pl.kernel(
        functools.partial(_gather_body, CW, nchunks, EW),
        out_type=[
            jax.ShapeDtypeStruct((E, H), dtype),
            jax.ShapeDtypeStruct((E, H), dtype),
        ],
        mesh=mesh,
        scratch_types=[
            pltpu.VMEM((2, CW), jnp.int32),
            pltpu.VMEM((2, CW), jnp.int32),
            pltpu.VMEM((2, CW, H), dtype),
            pltpu.VMEM((2, CW, H), dtype),
            pltpu.SemaphoreType.DMA((2,)),
            pltpu.SemaphoreType.DMA((2,)),
        ],
    )


_WL = 32


def _scatter_body(CW, nchunks, EW, NP, TW, vals, cid, zS, rseq, sp, shS, idx, ubuf, semadd):
    c = lax.axis_index("c")
    s = lax.axis_index("s")
    wid = s * _NC + c
    rows = NP // _NS
    r0 = s * rows

    def set_seq(j, b):
        pltpu.sync_copy(rseq.at[pl.ds(r0 + j * CW, CW)], idx.at[b])

    pltpu.sync_copy(zS.at[pl.ds(0, CW)], ubuf.at[0])
    for j in range(rows // CW):
        set_seq(j, 0)
        pltpu.sync_copy(ubuf.at[0], shS.at[idx.at[0]])
    plsc.subcore_barrier()

    base0 = wid * EW

    def body(i, carry):
        b = lax.rem(i, 2)
        @pl.when(i >= 2)
        def _():
            pltpu.make_async_copy(ubuf.at[b], shS.at[idx.at[b]],
                                  semadd.at[b]).wait()

        base = base0 + i * CW
        pltpu.sync_copy(cid.at[pl.ds(base, CW)], idx.at[b])
        pltpu.sync_copy(vals.at[pl.ds(base, CW)], ubuf.at[b])
        pltpu.async_copy(ubuf.at[b], shS.at[idx.at[b]], semadd.at[b], add=True)
        return carry

    lax.fori_loop(0, nchunks, body, 0)
    for b in range(2):
        pltpu.make_async_copy(ubuf.at[b], shS.at[idx.at[b]], semadd.at[b]).wait()
    plsc.subcore_barrier()
    for j in range(rows // CW):
        set_seq(j, 0)
        pltpu.sync_copy(shS.at[idx.at[0]], ubuf.at[0])
        pltpu.sync_copy(ubuf.at[0], sp.at[c, pl.ds(r0 + j * CW, CW)])


def _make_scatter(E, NP, TW):
    CW = 80
    EW = E // _NW
    nchunks = EW // CW
    mesh = plsc.VectorSubcoreMesh(core_axis_name="c", subcore_axis_name="s")
    return pl.kernel(
        functools.partial(_scatter_body, CW, nchunks, EW, NP, TW),
        out_type=jax.ShapeDtypeStruct((_NC, NP, TW), jnp.float32),
        mesh=mesh,
        scratch_types=[
            pltpu.VMEM_SHARED((NP, TW), jnp.float32),
            pltpu.VMEM((2, CW), jnp.int32),
            pltpu.VMEM((2, CW, TW), jnp.float32),
            pltpu.SemaphoreType.DMA((2,)),
        ],
    )


def _logits_body(ea, gc, gd, wc, we, wd, b1, w2, b2, w3, b3, lo, gmax):
    t = jnp.dot(gc[...], wc[...], preferred_element_type=jnp.float32)
    t = t + jnp.dot(ea[...], we[...], preferred_element_type=jnp.float32)
    t = t + jnp.dot(gd[...], wd[...], preferred_element_type=jnp.float32)
    t = jnp.maximum(t + b1[...], 0.0)
    t = jnp.maximum(jnp.dot(t, w2[...], preferred_element_type=jnp.float32) + b2[...], 0.0)
    l = jnp.dot(t, w3[...], preferred_element_type=jnp.float32) + b3[...]
    lo[...] = l
    m = jnp.max(l)

    @pl.when(pl.program_id(0) == 0)
    def _init():
        gmax[...] = jnp.full((8, 128), m, jnp.float32)

    @pl.when(pl.program_id(0) > 0)
    def _acc():
        gmax[...] = jnp.maximum(gmax[...], m)


def _gelu(x):
    return x * 0.5 * (1.0 + lax.erf(x * (1.0 / math.sqrt(2.0))))


def _values_body(ea, gd, lo, gmax, w1e, w1d, b1, w2, b2, w3, b3, K, u_out, w_out):
    v = jnp.dot(ea[...], w1e[...], preferred_element_type=jnp.float32)
    v = v + jnp.dot(gd[...], w1d[...], preferred_element_type=jnp.float32)
    v = _gelu(v + b1[...])
    v = _gelu(jnp.dot(v, w2[...], preferred_element_type=jnp.float32) + b2[...])
    V = jnp.dot(v, w3[...], preferred_element_type=jnp.float32) + b3[...]
    w = jnp.exp(lo[...] - gmax[0:1, :_LP])
    wrep = jnp.dot(w, K[...], preferred_element_type=jnp.float32,
                   precision=lax.Precision.HIGHEST)
    w_out[...] = wrep
    u_out[...] = V * wrep


def _final_body(sp, wp, wo, out):
    S = sp[0] + sp[1]
    W = wp[0] + wp[1]
    W = jnp.where(W > 0.0, W, 1.0)
    out[...] = jnp.dot(S / W, wo[...], preferred_element_type=jnp.float32)


def kernel(h_V, edge_index, edge_attr, wv1, bv1, wv2, bv2, wv3, bv3,
           wb1, bb1, wb2, bb2, wb3, bb3, wo):
    N, H = h_V.shape
    E = edge_index.shape[1]
    NH = wb3.shape[1]
    d = H // NH
    f32 = jnp.float32

    center = edge_index[0]
    dst = edge_index[1]
    wb1c, wb1e, wb1d = wb1[:H], wb1[H:2 * H], wb1[2 * H:]
    wv1e, wv1d = wv1[:H], wv1[H:]
    scale = 1.0 / math.sqrt(d)
    wb3p = jnp.concatenate([wb3 * scale, jnp.zeros((H, _LP - NH), f32)], axis=1)
    bb3p = jnp.concatenate([bb3 * scale, jnp.full((_LP - NH,), _NEG, f32)])
    K = jnp.concatenate(
        [jnp.kron(jnp.eye(NH, dtype=f32), jnp.ones((1, d), f32)),
         jnp.zeros((_LP - NH, H), f32)], axis=0)
    b1b = bb1.reshape(1, H)
    b2b = bb2.reshape(1, H)
    b3b = bb3p.reshape(1, _LP)
    v1b = bv1.reshape(1, H)
    v2b = bv2.reshape(1, H)
    v3b = bv3.reshape(1, H)

    gc, gd = _make_gather(E, N, H, f32)(h_V, center, dst)

    BE = 5000
    grid = E // BE
    row_spec = pl.BlockSpec((BE, H), lambda i: (i, 0))
    full = lambda shape: pl.BlockSpec(shape, lambda i: tuple(0 for _ in shape))
    logits, gmax = pl.pallas_call(
        _logits_body,
        grid=(grid,),
        in_specs=[row_spec, row_spec, row_spec,
                  full((H, H)), full((H, H)), full((H, H)), full((1, H)),
                  full((H, H)), full((1, H)), full((H, _LP)), full((1, _LP))],
        out_specs=[pl.BlockSpec((BE, _LP), lambda i: (i, 0)),
                   pl.BlockSpec((8, 128), lambda i: (0, 0))],
        out_shape=[jax.ShapeDtypeStruct((E, _LP), f32),
                   jax.ShapeDtypeStruct((8, 128), f32)],
    )(edge_attr, gc, gd, wb1c, wb1e, wb1d, b1b, wb2, b2b, wb3p, b3b)

    u, w16 = pl.pallas_call(
        _values_body,
        grid=(grid,),
        in_specs=[row_spec, row_spec,
                  pl.BlockSpec((BE, _LP), lambda i: (i, 0)), full((8, 128)),
                  full((H, H)), full((H, H)), full((1, H)),
                  full((H, H)), full((1, H)), full((H, H)), full((1, H)),
                  full((_LP, H))],
        out_specs=[row_spec, row_spec],
        out_shape=[jax.ShapeDtypeStruct((E, H), f32),
                   jax.ShapeDtypeStruct((E, H), f32)],
    )(edge_attr, gd, logits, gmax, wv1e, wv1d, v1b, wv2, v2b, wv3, v3b, K)

    NP = ((N + 80 * _NS - 1) // (80 * _NS)) * (80 * _NS)
    zS = jnp.zeros((NP, H), f32)
    rseq = jnp.arange(NP, dtype=jnp.int32)
    scatter = _make_scatter(E, NP, H)
    sp = scatter(u, center, zS, rseq)
    wp = scatter(w16, center, zS, rseq)

    BN = NP // 8
    out = pl.pallas_call(
        _final_body,
        grid=(NP // BN,),
        in_specs=[pl.BlockSpec((_NC, BN, H), lambda i: (0, i, 0)),
                  pl.BlockSpec((_NC, BN, H), lambda i: (0, i, 0)),
                  full((H, H))],
        out_specs=pl.BlockSpec((BN, H), lambda i: (i, 0)),
        out_shape=jax.ShapeDtypeStruct((NP, H), f32),
    )(sp, wp, wo)
    return out[:N]


# speedup vs baseline: 1.1134x; 1.1134x over previous
import functools
import math

import jax
import jax.numpy as jnp
from jax import lax
from jax.experimental import pallas as pl
from jax.experimental.pallas import tpu as pltpu
from jax.experimental.pallas import tpu_sc as plsc

_NC = 2
_NS = 16
_NW = _NC * _NS

_LP = 16
_NEG = -1e30


def _gather_body(CW, nchunks, EW, hv, cid, did, gc, gd, idxc, idxd, rowsc, rowsd, semc, semd):
    wid = lax.axis_index("s") * _NC + lax.axis_index("c")
    base0 = wid * EW

    def fetch(i, b):
        base = base0 + i * CW
        pltpu.sync_copy(cid.at[pl.ds(base, CW)], idxc.at[b])
        pltpu.sync_copy(did.at[pl.ds(base, CW)], idxd.at[b])
        pltpu.async_copy(hv.at[idxc.at[b]], rowsc.at[b], semc.at[b])
        pltpu.async_copy(hv.at[idxd.at[b]], rowsd.at[b], semd.at[b])

    fetch(0, 0)

    def body(i, carry):
        b = lax.rem(i, 2)
        @pl.when(i + 1 < nchunks)
        def _():
            fetch(i + 1, 1 - b)

        pltpu.make_async_copy(hv.at[idxc.at[b]], rowsc.at[b], semc.at[b]).wait()
        pltpu.make_async_copy(hv.at[idxd.at[b]], rowsd.at[b], semd.at[b]).wait()
        base = base0 + i * CW
        pltpu.sync_copy(rowsc.at[b], gc.at[pl.ds(base, CW)])
        pltpu.sync_copy(rowsd.at[b], gd.at[pl.ds(base, CW)])
        return carry

    lax.fori_loop(0, nchunks, body, 0)


def _make_gather(E, N, H, dtype):
    CW = 80
    EW = E // _NW
    nchunks = EW // CW
    mesh = plsc.VectorSubcoreMesh(core_axis_name="c", subcore_axis_name="s")
    return pl.kernel(
        functools.partial(_gather_body, CW, nchunks, EW),
        out_type=[
            jax.ShapeDtypeStruct((E, H), dtype),
            jax.ShapeDtypeStruct((E, H), dtype),
        ],
        mesh=mesh,
        scratch_types=[
            pltpu.VMEM((2, CW), jnp.int32),
            pltpu.VMEM((2, CW), jnp.int32),
            pltpu.VMEM((2, CW, H), dtype),
            pltpu.VMEM((2, CW, H), dtype),
            pltpu.SemaphoreType.DMA((2,)),
            pltpu.SemaphoreType.DMA((2,)),
        ],
    )


_WL = 32


def _scatter_body(CW, nchunks, EW, NP, TW, vals, cid, zS, rseq, sp, shS, idx, ubuf, semadd):
    c = lax.axis_index("c")
    s = lax.axis_index("s")
    wid = s * _NC + c
    rows = NP // _NS
    r0 = s * rows

    def set_seq(j, b):
        pltpu.sync_copy(rseq.at[pl.ds(r0 + j * CW, CW)], idx.at[b])

    pltpu.sync_copy(zS.at[pl.ds(0, CW)], ubuf.at[0])
    for j in range(rows // CW):
        set_seq(j, 0)
        pltpu.sync_copy(ubuf.at[0], shS.at[idx.at[0]])
    plsc.subcore_barrier()

    base0 = wid * EW

    def body(i, carry):
        b = lax.rem(i, 2)
        @pl.when(i >= 2)
        def _():
            pltpu.make_async_copy(ubuf.at[b], shS.at[idx.at[b]],
                                  semadd.at[b]).wait()

        base = base0 + i * CW
        pltpu.sync_copy(cid.at[pl.ds(base, CW)], idx.at[b])
        pltpu.sync_copy(vals.at[pl.ds(base, CW)], ubuf.at[b])
        pltpu.async_copy(ubuf.at[b], shS.at[idx.at[b]], semadd.at[b], add=True)
        return carry

    lax.fori_loop(0, nchunks, body, 0)
    for b in range(2):
        pltpu.make_async_copy(ubuf.at[b], shS.at[idx.at[b]], semadd.at[b]).wait()
    plsc.subcore_barrier()
    for j in range(rows // CW):
        set_seq(j, 0)
        pltpu.sync_copy(shS.at[idx.at[0]], ubuf.at[0])
        pltpu.sync_copy(ubuf.at[0], sp.at[c, pl.ds(r0 + j * CW, CW)])


def _make_scatter(E, NP, TW):
    CW = 80
    EW = E // _NW
    nchunks = EW // CW
    mesh = plsc.VectorSubcoreMesh(core_axis_name="c", subcore_axis_name="s")
    return pl.kernel(
        functools.partial(_scatter_body, CW, nchunks, EW, NP, TW),
        out_type=jax.ShapeDtypeStruct((_NC, NP, TW), jnp.float32),
        mesh=mesh,
        scratch_types=[
            pltpu.VMEM_SHARED((NP, TW), jnp.float32),
            pltpu.VMEM((2, CW), jnp.int32),
            pltpu.VMEM((2, CW, TW), jnp.float32),
            pltpu.SemaphoreType.DMA((2,)),
        ],
    )


def _logits_body(ea, gc, gd, wc, we, wd, b1, w2, b2, w3, b3, lo, gmax):
    t = jnp.dot(gc[...], wc[...], preferred_element_type=jnp.float32)
    t = t + jnp.dot(ea[...], we[...], preferred_element_type=jnp.float32)
    t = t + jnp.dot(gd[...], wd[...], preferred_element_type=jnp.float32)
    t = jnp.maximum(t + b1[...], 0.0)
    t = jnp.maximum(jnp.dot(t, w2[...], preferred_element_type=jnp.float32) + b2[...], 0.0)
    l = jnp.dot(t, w3[...], preferred_element_type=jnp.float32) + b3[...]
    lo[...] = l
    m = jnp.max(l)

    @pl.when(pl.program_id(0) == 0)
    def _init():
        gmax[...] = jnp.full((8, 128), m, jnp.float32)

    @pl.when(pl.program_id(0) > 0)
    def _acc():
        gmax[...] = jnp.maximum(gmax[...], m)


def _gelu(x):
    return x * 0.5 * (1.0 + lax.erf(x * (1.0 / math.sqrt(2.0))))


def _wrep_body(lo, gmax, K, w_out):
    w = jnp.exp(lo[...] - gmax[0:1, :_LP])
    w_out[...] = jnp.dot(w, K[...], preferred_element_type=jnp.float32,
                         precision=lax.Precision.HIGHEST)


def _values_body(ea, gd, lo, gmax, w1e, w1d, b1, w2, b2, w3, b3, K, u_out):
    v = jnp.dot(ea[...], w1e[...], preferred_element_type=jnp.float32)
    v = v + jnp.dot(gd[...], w1d[...], preferred_element_type=jnp.float32)
    v = _gelu(v + b1[...])
    v = _gelu(jnp.dot(v, w2[...], preferred_element_type=jnp.float32) + b2[...])
    V = jnp.dot(v, w3[...], preferred_element_type=jnp.float32) + b3[...]
    w = jnp.exp(lo[...] - gmax[0:1, :_LP])
    wrep = jnp.dot(w, K[...], preferred_element_type=jnp.float32,
                   precision=lax.Precision.HIGHEST)
    u_out[...] = V * wrep


def _final_body(sp, wp, wo, out):
    S = sp[0] + sp[1]
    W = wp[0] + wp[1]
    W = jnp.where(W > 0.0, W, 1.0)
    out[...] = jnp.dot(S / W, wo[...], preferred_element_type=jnp.float32)


def kernel(h_V, edge_index, edge_attr, wv1, bv1, wv2, bv2, wv3, bv3,
           wb1, bb1, wb2, bb2, wb3, bb3, wo):
    N, H = h_V.shape
    E = edge_index.shape[1]
    NH = wb3.shape[1]
    d = H // NH
    f32 = jnp.float32

    center = edge_index[0]
    dst = edge_index[1]
    wb1c, wb1e, wb1d = wb1[:H], wb1[H:2 * H], wb1[2 * H:]
    wv1e, wv1d = wv1[:H], wv1[H:]
    scale = 1.0 / math.sqrt(d)
    wb3p = jnp.concatenate([wb3 * scale, jnp.zeros((H, _LP - NH), f32)], axis=1)
    bb3p = jnp.concatenate([bb3 * scale, jnp.full((_LP - NH,), _NEG, f32)])
    K = jnp.concatenate(
        [jnp.kron(jnp.eye(NH, dtype=f32), jnp.ones((1, d), f32)),
         jnp.zeros((_LP - NH, H), f32)], axis=0)
    b1b = bb1.reshape(1, H)
    b2b = bb2.reshape(1, H)
    b3b = bb3p.reshape(1, _LP)
    v1b = bv1.reshape(1, H)
    v2b = bv2.reshape(1, H)
    v3b = bv3.reshape(1, H)

    gc, gd = _make_gather(E, N, H, f32)(h_V, center, dst)

    BE = 4000
    grid = E // BE
    row_spec = pl.BlockSpec((BE, H), lambda i: (i, 0))
    full = lambda shape: pl.BlockSpec(shape, lambda i: tuple(0 for _ in shape))
    logits, gmax = pl.pallas_call(
        _logits_body,
        grid=(grid,),
        in_specs=[row_spec, row_spec, row_spec,
                  full((H, H)), full((H, H)), full((H, H)), full((1, H)),
                  full((H, H)), full((1, H)), full((H, _LP)), full((1, _LP))],
        out_specs=[pl.BlockSpec((BE, _LP), lambda i: (i, 0)),
                   pl.BlockSpec((8, 128), lambda i: (0, 0))],
        out_shape=[jax.ShapeDtypeStruct((E, _LP), f32),
                   jax.ShapeDtypeStruct((8, 128), f32)],
    )(edge_attr, gc, gd, wb1c, wb1e, wb1d, b1b, wb2, b2b, wb3p, b3b)

    w16 = pl.pallas_call(
        _wrep_body,
        grid=(grid,),
        in_specs=[pl.BlockSpec((BE, _LP), lambda i: (i, 0)), full((8, 128)),
                  full((_LP, H))],
        out_specs=row_spec,
        out_shape=jax.ShapeDtypeStruct((E, H), f32),
    )(logits, gmax, K)

    u = pl.pallas_call(
        _values_body,
        grid=(grid,),
        in_specs=[row_spec, row_spec,
                  pl.BlockSpec((BE, _LP), lambda i: (i, 0)), full((8, 128)),
                  full((H, H)), full((H, H)), full((1, H)),
                  full((H, H)), full((1, H)), full((H, H)), full((1, H)),
                  full((_LP, H))],
        out_specs=row_spec,
        out_shape=jax.ShapeDtypeStruct((E, H), f32),
    )(edge_attr, gd, logits, gmax, wv1e, wv1d, v1b, wv2, v2b, wv3, v3b, K)

    NP = ((N + 80 * _NS - 1) // (80 * _NS)) * (80 * _NS)
    zS = jnp.zeros((NP, H), f32)
    rseq = jnp.arange(NP, dtype=jnp.int32)
    scatter = _make_scatter(E, NP, H)
    wp = scatter(w16, center, zS, rseq)
    sp = scatter(u, center, zS, rseq)

    BN = NP // 8
    out = pl.pallas_call(
        _final_body,
        grid=(NP // BN,),
        in_specs=[pl.BlockSpec((_NC, BN, H), lambda i: (0, i, 0)),
                  pl.BlockSpec((_NC, BN, H), lambda i: (0, i, 0)),
                  full((H, H))],
        out_specs=pl.BlockSpec((BN, H), lambda i: (i, 0)),
        out_shape=jax.ShapeDtypeStruct((NP, H), f32),
    )(sp, wp, wo)
    return out[:N]


# speedup vs baseline: 1.2222x; 1.0977x over previous
import functools
import math

import jax
import jax.numpy as jnp
from jax import lax
from jax.experimental import pallas as pl
from jax.experimental.pallas import tpu as pltpu
from jax.experimental.pallas import tpu_sc as plsc

_NC = 2
_NS = 16
_NW = _NC * _NS

_LP = 16
_NEG = -1e30


def _gather_body(CW, nchunks, EW, hv, cid, did, gc, gd, idxc, idxd, rowsc, rowsd, semc, semd):
    wid = lax.axis_index("s") * _NC + lax.axis_index("c")
    base0 = wid * EW

    def fetch(i, b):
        base = base0 + i * CW
        pltpu.sync_copy(cid.at[pl.ds(base, CW)], idxc.at[b])
        pltpu.sync_copy(did.at[pl.ds(base, CW)], idxd.at[b])
        pltpu.async_copy(hv.at[idxc.at[b]], rowsc.at[b], semc.at[b])
        pltpu.async_copy(hv.at[idxd.at[b]], rowsd.at[b], semd.at[b])

    fetch(0, 0)

    def body(i, carry):
        b = lax.rem(i, 2)
        @pl.when(i + 1 < nchunks)
        def _():
            fetch(i + 1, 1 - b)

        pltpu.make_async_copy(hv.at[idxc.at[b]], rowsc.at[b], semc.at[b]).wait()
        pltpu.make_async_copy(hv.at[idxd.at[b]], rowsd.at[b], semd.at[b]).wait()
        base = base0 + i * CW
        pltpu.sync_copy(rowsc.at[b], gc.at[pl.ds(base, CW)])
        pltpu.sync_copy(rowsd.at[b], gd.at[pl.ds(base, CW)])
        return carry

    lax.fori_loop(0, nchunks, body, 0)


def _make_gather(E, N, H, dtype):
    CW = 80
    EW = E // _NW
    nchunks = EW // CW
    mesh = plsc.VectorSubcoreMesh(core_axis_name="c", subcore_axis_name="s")
    return pl.kernel(
        functools.partial(_gather_body, CW, nchunks, EW),
        out_type=[
            jax.ShapeDtypeStruct((E, H), dtype),
            jax.ShapeDtypeStruct((E, H), dtype),
        ],
        mesh=mesh,
        scratch_types=[
            pltpu.VMEM((2, CW), jnp.int32),
            pltpu.VMEM((2, CW), jnp.int32),
            pltpu.VMEM((2, CW, H), dtype),
            pltpu.VMEM((2, CW, H), dtype),
            pltpu.SemaphoreType.DMA((2,)),
            pltpu.SemaphoreType.DMA((2,)),
        ],
    )


_WL = 32


def _scatter_body(CW, nchunks, EW, NP, TW, vals, cid, zS, rseq, sp, shS, idx, ubuf, semadd, semld, semld2):
    c = lax.axis_index("c")
    s = lax.axis_index("s")
    wid = s * _NC + c
    rows = NP // _NS
    r0 = s * rows

    def set_seq(j, b):
        pltpu.sync_copy(rseq.at[pl.ds(r0 + j * CW, CW)], idx.at[b])

    pltpu.sync_copy(zS.at[pl.ds(0, CW)], ubuf.at[0])
    for j in range(rows // CW):
        set_seq(j, 0)
        pltpu.sync_copy(ubuf.at[0], shS.at[idx.at[0]])
    plsc.subcore_barrier()

    base0 = wid * EW

    def body(i, carry):
        b = lax.rem(i, 2)
        @pl.when(i >= 2)
        def _():
            pltpu.make_async_copy(ubuf.at[b], shS.at[idx.at[b]],
                                  semadd.at[b]).wait()

        base = base0 + i * CW
        ci = pltpu.async_copy(cid.at[pl.ds(base, CW)], idx.at[b], semld.at[b])
        vi = pltpu.async_copy(vals.at[pl.ds(base, CW)], ubuf.at[b],
                              semld2.at[b])
        ci.wait()
        vi.wait()
        pltpu.async_copy(ubuf.at[b], shS.at[idx.at[b]], semadd.at[b], add=True)
        return carry

    lax.fori_loop(0, nchunks, body, 0)
    for b in range(2):
        pltpu.make_async_copy(ubuf.at[b], shS.at[idx.at[b]], semadd.at[b]).wait()
    plsc.subcore_barrier()
    for j in range(rows // CW):
        set_seq(j, 0)
        pltpu.sync_copy(shS.at[idx.at[0]], ubuf.at[0])
        pltpu.sync_copy(ubuf.at[0], sp.at[c, pl.ds(r0 + j * CW, CW)])


def _make_scatter(E, NP, TW):
    CW = 80
    EW = E // _NW
    nchunks = EW // CW
    mesh = plsc.VectorSubcoreMesh(core_axis_name="c", subcore_axis_name="s")
    return pl.kernel(
        functools.partial(_scatter_body, CW, nchunks, EW, NP, TW),
        out_type=jax.ShapeDtypeStruct((_NC, NP, TW), jnp.float32),
        mesh=mesh,
        scratch_types=[
            pltpu.VMEM_SHARED((NP, TW), jnp.float32),
            pltpu.VMEM((2, CW), jnp.int32),
            pltpu.VMEM((2, CW, TW), jnp.float32),
            pltpu.SemaphoreType.DMA((2,)),
            pltpu.SemaphoreType.DMA((2,)),
            pltpu.SemaphoreType.DMA((2,)),
        ],
    )


def _logits_body(ea, gc, gd, wc, we, wd, b1, w2, b2, w3, b3, lo, gmax):
    t = jnp.dot(gc[...], wc[...], preferred_element_type=jnp.float32)
    t = t + jnp.dot(ea[...], we[...], preferred_element_type=jnp.float32)
    t = t + jnp.dot(gd[...], wd[...], preferred_element_type=jnp.float32)
    t = jnp.maximum(t + b1[...], 0.0)
    t = jnp.maximum(jnp.dot(t, w2[...], preferred_element_type=jnp.float32) + b2[...], 0.0)
    l = jnp.dot(t, w3[...], preferred_element_type=jnp.float32) + b3[...]
    lo[...] = l
    m = jnp.max(l)

    @pl.when(pl.program_id(0) == 0)
    def _init():
        gmax[...] = jnp.full((8, 128), m, jnp.float32)

    @pl.when(pl.program_id(0) > 0)
    def _acc():
        gmax[...] = jnp.maximum(gmax[...], m)


def _gelu(x):
    return x * 0.5 * (1.0 + lax.erf(x * (1.0 / math.sqrt(2.0))))


def _values_body(ea, gd, lo, gmax, w1e, w1d, b1, w2, b2, w3, b3, K, u_out, w_out):
    v = jnp.dot(ea[...], w1e[...], preferred_element_type=jnp.float32)
    v = v + jnp.dot(gd[...], w1d[...], preferred_element_type=jnp.float32)
    v = _gelu(v + b1[...])
    v = _gelu(jnp.dot(v, w2[...], preferred_element_type=jnp.float32) + b2[...])
    V = jnp.dot(v, w3[...], preferred_element_type=jnp.float32) + b3[...]
    w = jnp.exp(lo[...] - gmax[0:1, :_LP])
    wrep = jnp.dot(w, K[...], preferred_element_type=jnp.float32,
                   precision=lax.Precision.HIGHEST)
    w_out[...] = wrep
    u_out[...] = V * wrep


def _final_body(sp, wp, wo, out):
    S = sp[0] + sp[1]
    W = wp[0] + wp[1]
    W = jnp.where(W > 0.0, W, 1.0)
    out[...] = jnp.dot(S / W, wo[...], preferred_element_type=jnp.float32)


def kernel(h_V, edge_index, edge_attr, wv1, bv1, wv2, bv2, wv3, bv3,
           wb1, bb1, wb2, bb2, wb3, bb3, wo):
    N, H = h_V.shape
    E = edge_index.shape[1]
    NH = wb3.shape[1]
    d = H // NH
    f32 = jnp.float32

    center = edge_index[0]
    dst = edge_index[1]
    wb1c, wb1e, wb1d = wb1[:H], wb1[H:2 * H], wb1[2 * H:]
    wv1e, wv1d = wv1[:H], wv1[H:]
    scale = 1.0 / math.sqrt(d)
    wb3p = jnp.concatenate([wb3 * scale, jnp.zeros((H, _LP - NH), f32)], axis=1)
    bb3p = jnp.concatenate([bb3 * scale, jnp.full((_LP - NH,), _NEG, f32)])
    K = jnp.concatenate(
        [jnp.kron(jnp.eye(NH, dtype=f32), jnp.ones((1, d), f32)),
         jnp.zeros((_LP - NH, H), f32)], axis=0)
    b1b = bb1.reshape(1, H)
    b2b = bb2.reshape(1, H)
    b3b = bb3p.reshape(1, _LP)
    v1b = bv1.reshape(1, H)
    v2b = bv2.reshape(1, H)
    v3b = bv3.reshape(1, H)

    gc, gd = _make_gather(E, N, H, f32)(h_V, center, dst)

    BE = 4000
    grid = E // BE
    row_spec = pl.BlockSpec((BE, H), lambda i: (i, 0))
    full = lambda shape: pl.BlockSpec(shape, lambda i: tuple(0 for _ in shape))
    logits, gmax = pl.pallas_call(
        _logits_body,
        grid=(grid,),
        in_specs=[row_spec, row_spec, row_spec,
                  full((H, H)), full((H, H)), full((H, H)), full((1, H)),
                  full((H, H)), full((1, H)), full((H, _LP)), full((1, _LP))],
        out_specs=[pl.BlockSpec((BE, _LP), lambda i: (i, 0)),
                   pl.BlockSpec((8, 128), lambda i: (0, 0))],
        out_shape=[jax.ShapeDtypeStruct((E, _LP), f32),
                   jax.ShapeDtypeStruct((8, 128), f32)],
    )(edge_attr, gc, gd, wb1c, wb1e, wb1d, b1b, wb2, b2b, wb3p, b3b)

    u, w16 = pl.pallas_call(
        _values_body,
        grid=(grid,),
        in_specs=[row_spec, row_spec,
                  pl.BlockSpec((BE, _LP), lambda i: (i, 0)), full((8, 128)),
                  full((H, H)), full((H, H)), full((1, H)),
                  full((H, H)), full((1, H)), full((H, H)), full((1, H)),
                  full((_LP, H))],
        out_specs=[row_spec, row_spec],
        out_shape=[jax.ShapeDtypeStruct((E, H), f32),
                   jax.ShapeDtypeStruct((E, H), f32)],
    )(edge_attr, gd, logits, gmax, wv1e, wv1d, v1b, wv2, v2b, wv3, v3b, K)

    NP = ((N + 80 * _NS - 1) // (80 * _NS)) * (80 * _NS)
    zS = jnp.zeros((NP, H), f32)
    rseq = jnp.arange(NP, dtype=jnp.int32)
    scatter = _make_scatter(E, NP, H)
    sp = scatter(u, center, zS, rseq)
    wp = scatter(w16, center, zS, rseq)

    BN = NP // 8
    out = pl.pallas_call(
        _final_body,
        grid=(NP // BN,),
        in_specs=[pl.BlockSpec((_NC, BN, H), lambda i: (0, i, 0)),
                  pl.BlockSpec((_NC, BN, H), lambda i: (0, i, 0)),
                  full((H, H))],
        out_specs=pl.BlockSpec((BN, H), lambda i: (i, 0)),
        out_shape=jax.ShapeDtypeStruct((NP, H), f32),
    )(sp, wp, wo)
    return out[:N]


# speedup vs baseline: 1.2661x; 1.0359x over previous
import functools
import math

import jax
import jax.numpy as jnp
from jax import lax
from jax.experimental import pallas as pl
from jax.experimental.pallas import tpu as pltpu
from jax.experimental.pallas import tpu_sc as plsc

_NC = 2
_NS = 16
_NW = _NC * _NS

_LP = 16
_NEG = -1e30


def _gather_body(CW, nchunks, EW, hv, cid, did, gc, gd, idxc, idxd, rowsc, rowsd, semc, semd, semi, semj):
    wid = lax.axis_index("s") * _NC + lax.axis_index("c")
    base0 = wid * EW

    def fetch(i, b):
        base = base0 + i * CW
        c1 = pltpu.async_copy(cid.at[pl.ds(base, CW)], idxc.at[b], semi.at[b])
        c2 = pltpu.async_copy(did.at[pl.ds(base, CW)], idxd.at[b], semj.at[b])
        c1.wait()
        c2.wait()
        pltpu.async_copy(hv.at[idxc.at[b]], rowsc.at[b], semc.at[b])
        pltpu.async_copy(hv.at[idxd.at[b]], rowsd.at[b], semd.at[b])

    fetch(0, 0)

    def body(i, carry):
        b = lax.rem(i, 2)
        @pl.when(i + 1 < nchunks)
        def _():
            fetch(i + 1, 1 - b)

        pltpu.make_async_copy(hv.at[idxc.at[b]], rowsc.at[b], semc.at[b]).wait()
        pltpu.make_async_copy(hv.at[idxd.at[b]], rowsd.at[b], semd.at[b]).wait()
        base = base0 + i * CW
        pltpu.sync_copy(rowsc.at[b], gc.at[pl.ds(base, CW)])
        pltpu.sync_copy(rowsd.at[b], gd.at[pl.ds(base, CW)])
        return carry

    lax.fori_loop(0, nchunks, body, 0)


def _make_gather(E, N, H, dtype):
    CW = 80
    EW = E // _NW
    nchunks = EW // CW
    mesh = plsc.VectorSubcoreMesh(core_axis_name="c", subcore_axis_name="s")
    return pl.kernel(
        functools.partial(_gather_body, CW, nchunks, EW),
        out_type=[
            jax.ShapeDtypeStruct((E, H), dtype),
            jax.ShapeDtypeStruct((E, H), dtype),
        ],
        mesh=mesh,
        scratch_types=[
            pltpu.VMEM((2, CW), jnp.int32),
            pltpu.VMEM((2, CW), jnp.int32),
            pltpu.VMEM((2, CW, H), dtype),
            pltpu.VMEM((2, CW, H), dtype),
            pltpu.SemaphoreType.DMA((2,)),
            pltpu.SemaphoreType.DMA((2,)),
            pltpu.SemaphoreType.DMA((2,)),
            pltpu.SemaphoreType.DMA((2,)),
        ],
    )


_WL = 32


def _scatter_body(CW, nchunks, EW, NP, TW, vals, cid, zS, rseq, sp, shS, idx, ubuf, semadd, semld, semld2):
    c = lax.axis_index("c")
    s = lax.axis_index("s")
    wid = s * _NC + c
    rows = NP // _NS
    r0 = s * rows

    def set_seq(j, b):
        pltpu.sync_copy(rseq.at[pl.ds(r0 + j * CW, CW)], idx.at[b])

    pltpu.sync_copy(zS.at[pl.ds(0, CW)], ubuf.at[0])
    for j in range(rows // CW):
        set_seq(j, 0)
        pltpu.sync_copy(ubuf.at[0], shS.at[idx.at[0]])
    plsc.subcore_barrier()

    base0 = wid * EW

    def body(i, carry):
        b = lax.rem(i, 2)
        @pl.when(i >= 2)
        def _():
            pltpu.make_async_copy(ubuf.at[b], shS.at[idx.at[b]],
                                  semadd.at[b]).wait()

        base = base0 + i * CW
        ci = pltpu.async_copy(cid.at[pl.ds(base, CW)], idx.at[b], semld.at[b])
        vi = pltpu.async_copy(vals.at[pl.ds(base, CW)], ubuf.at[b],
                              semld2.at[b])
        ci.wait()
        vi.wait()
        pltpu.async_copy(ubuf.at[b], shS.at[idx.at[b]], semadd.at[b], add=True)
        return carry

    lax.fori_loop(0, nchunks, body, 0)
    for b in range(2):
        pltpu.make_async_copy(ubuf.at[b], shS.at[idx.at[b]], semadd.at[b]).wait()
    plsc.subcore_barrier()
    for j in range(rows // CW):
        set_seq(j, 0)
        pltpu.sync_copy(shS.at[idx.at[0]], ubuf.at[0])
        pltpu.sync_copy(ubuf.at[0], sp.at[c, pl.ds(r0 + j * CW, CW)])


def _make_scatter(E, NP, TW):
    CW = 80
    EW = E // _NW
    nchunks = EW // CW
    mesh = plsc.VectorSubcoreMesh(core_axis_name="c", subcore_axis_name="s")
    return pl.kernel(
        functools.partial(_scatter_body, CW, nchunks, EW, NP, TW),
        out_type=jax.ShapeDtypeStruct((_NC, NP, TW), jnp.float32),
        mesh=mesh,
        scratch_types=[
            pltpu.VMEM_SHARED((NP, TW), jnp.float32),
            pltpu.VMEM((2, CW), jnp.int32),
            pltpu.VMEM((2, CW, TW), jnp.float32),
            pltpu.SemaphoreType.DMA((2,)),
            pltpu.SemaphoreType.DMA((2,)),
            pltpu.SemaphoreType.DMA((2,)),
        ],
    )


def _logits_body(ea, gc, gd, wc, we, wd, b1, w2, b2, w3, b3, lo, gmax):
    t = jnp.dot(gc[...], wc[...], preferred_element_type=jnp.float32)
    t = t + jnp.dot(ea[...], we[...], preferred_element_type=jnp.float32)
    t = t + jnp.dot(gd[...], wd[...], preferred_element_type=jnp.float32)
    t = jnp.maximum(t + b1[...], 0.0)
    t = jnp.maximum(jnp.dot(t, w2[...], preferred_element_type=jnp.float32) + b2[...], 0.0)
    l = jnp.dot(t, w3[...], preferred_element_type=jnp.float32) + b3[...]
    lo[...] = l
    m = jnp.max(l)

    @pl.when(pl.program_id(0) == 0)
    def _init():
        gmax[...] = jnp.full((8, 128), m, jnp.float32)

    @pl.when(pl.program_id(0) > 0)
    def _acc():
        gmax[...] = jnp.maximum(gmax[...], m)


def _gelu(x):
    return x * 0.5 * (1.0 + lax.erf(x * (1.0 / math.sqrt(2.0))))


def _values_body(ea, gd, lo, gmax, w1e, w1d, b1, w2, b2, w3, b3, K, u_out, w_out):
    v = jnp.dot(ea[...], w1e[...], preferred_element_type=jnp.float32)
    v = v + jnp.dot(gd[...], w1d[...], preferred_element_type=jnp.float32)
    v = _gelu(v + b1[...])
    v = _gelu(jnp.dot(v, w2[...], preferred_element_type=jnp.float32) + b2[...])
    V = jnp.dot(v, w3[...], preferred_element_type=jnp.float32) + b3[...]
    w = jnp.exp(lo[...] - gmax[0:1, :_LP])
    wrep = jnp.dot(w, K[...], preferred_element_type=jnp.float32,
                   precision=lax.Precision.HIGHEST)
    w_out[...] = wrep
    u_out[...] = V * wrep


def _final_body(sp, wp, wo, out):
    S = sp[0] + sp[1]
    W = wp[0] + wp[1]
    W = jnp.where(W > 0.0, W, 1.0)
    out[...] = jnp.dot(S / W, wo[...], preferred_element_type=jnp.float32)


def kernel(h_V, edge_index, edge_attr, wv1, bv1, wv2, bv2, wv3, bv3,
           wb1, bb1, wb2, bb2, wb3, bb3, wo):
    N, H = h_V.shape
    E = edge_index.shape[1]
    NH = wb3.shape[1]
    d = H // NH
    f32 = jnp.float32

    center = edge_index[0]
    dst = edge_index[1]
    wb1c, wb1e, wb1d = wb1[:H], wb1[H:2 * H], wb1[2 * H:]
    wv1e, wv1d = wv1[:H], wv1[H:]
    scale = 1.0 / math.sqrt(d)
    wb3p = jnp.concatenate([wb3 * scale, jnp.zeros((H, _LP - NH), f32)], axis=1)
    bb3p = jnp.concatenate([bb3 * scale, jnp.full((_LP - NH,), _NEG, f32)])
    K = jnp.concatenate(
        [jnp.kron(jnp.eye(NH, dtype=f32), jnp.ones((1, d), f32)),
         jnp.zeros((_LP - NH, H), f32)], axis=0)
    b1b = bb1.reshape(1, H)
    b2b = bb2.reshape(1, H)
    b3b = bb3p.reshape(1, _LP)
    v1b = bv1.reshape(1, H)
    v2b = bv2.reshape(1, H)
    v3b = bv3.reshape(1, H)

    gc, gd = _make_gather(E, N, H, f32)(h_V, center, dst)

    BE = 4000
    grid = E // BE
    row_spec = pl.BlockSpec((BE, H), lambda i: (i, 0))
    full = lambda shape: pl.BlockSpec(shape, lambda i: tuple(0 for _ in shape))
    logits, gmax = pl.pallas_call(
        _logits_body,
        grid=(grid,),
        in_specs=[row_spec, row_spec, row_spec,
                  full((H, H)), full((H, H)), full((H, H)), full((1, H)),
                  full((H, H)), full((1, H)), full((H, _LP)), full((1, _LP))],
        out_specs=[pl.BlockSpec((BE, _LP), lambda i: (i, 0)),
                   pl.BlockSpec((8, 128), lambda i: (0, 0))],
        out_shape=[jax.ShapeDtypeStruct((E, _LP), f32),
                   jax.ShapeDtypeStruct((8, 128), f32)],
    )(edge_attr, gc, gd, wb1c, wb1e, wb1d, b1b, wb2, b2b, wb3p, b3b)

    u, w16 = pl.pallas_call(
        _values_body,
        grid=(grid,),
        in_specs=[row_spec, row_spec,
                  pl.BlockSpec((BE, _LP), lambda i: (i, 0)), full((8, 128)),
                  full((H, H)), full((H, H)), full((1, H)),
                  full((H, H)), full((1, H)), full((H, H)), full((1, H)),
                  full((_LP, H))],
        out_specs=[row_spec, row_spec],
        out_shape=[jax.ShapeDtypeStruct((E, H), f32),
                   jax.ShapeDtypeStruct((E, H), f32)],
    )(edge_attr, gd, logits, gmax, wv1e, wv1d, v1b, wv2, v2b, wv3, v3b, K)

    NP = ((N + 80 * _NS - 1) // (80 * _NS)) * (80 * _NS)
    zS = jnp.zeros((NP, H), f32)
    rseq = jnp.arange(NP, dtype=jnp.int32)
    scatter = _make_scatter(E, NP, H)
    sp = scatter(u, center, zS, rseq)
    wp = scatter(w16, center, zS, rseq)

    BN = NP // 8
    out = pl.pallas_call(
        _final_body,
        grid=(NP // BN,),
        in_specs=[pl.BlockSpec((_NC, BN, H), lambda i: (0, i, 0)),
                  pl.BlockSpec((_NC, BN, H), lambda i: (0, i, 0)),
                  full((H, H))],
        out_specs=pl.BlockSpec((BN, H), lambda i: (i, 0)),
        out_shape=jax.ShapeDtypeStruct((NP, H), f32),
    )(sp, wp, wo)
    return out[:N]
